# baseline (speedup 1.0000x reference)
import functools

import jax
import jax.numpy as jnp
from jax import lax
from jax.experimental import pallas as pl
from jax.experimental.pallas import tpu as pltpu
from jax.experimental.pallas import tpu_sc as plsc

_NC = 2
_NS = 16
_L = 16
_NW = _NC * _NS


def _sc_mesh():
    return plsc.VectorSubcoreMesh(
        core_axis_name="c", subcore_axis_name="s",
        num_cores=_NC, num_subcores=_NS)


def _chunk_size(per_tile):
    for c in range(128, 0, -16):
        if per_tile % c == 0:
            return c
    raise ValueError(f"per-tile edge count {per_tile} not divisible by 16")


def _pad_nodes(n):
    blk = _NS * 8
    return ((n + blk - 1) // blk) * blk


def _sc_skeleton(table, idx):
    v, d = table.shape
    b = idx.shape[0]
    bpw = b // _NW

    @functools.partial(
        pl.kernel,
        out_type=jax.ShapeDtypeStruct((b, d), jnp.float32),
        mesh=_sc_mesh(),
        scratch_types=[
            pltpu.VMEM((bpw,), jnp.int32),
            pltpu.VMEM((bpw, d), jnp.float32),
            pltpu.VMEM_SHARED((b, d), jnp.float32),
            pltpu.SemaphoreType.DMA,
        ],
    )
    def k(table_hbm, idx_hbm, out_hbm, idx_v, rows_v, sh, sem):
        s = lax.axis_index("s")
        wid = s * _NC + lax.axis_index("c")
        base = wid * bpw
        sbase = s * (b // _NS)
        def fill(i, _):
            for kk in range(d // _L):
                rows_v[i, pl.ds(kk * _L, _L)] = jnp.zeros((_L,), jnp.float32)
            return 0
        lax.fori_loop(0, bpw, fill, 0)
        plsc.subcore_barrier()
        pltpu.sync_copy(idx_hbm.at[pl.ds(base, bpw)], idx_v)
        pltpu.async_copy(table_hbm.at[idx_v], rows_v, sem).wait()
        pltpu.sync_copy(rows_v, sh.at[pl.ds(sbase, bpw)])
        pltpu.sync_copy(sh.at[pl.ds(sbase, bpw)], rows_v)
        pltpu.sync_copy(rows_v, out_hbm.at[pl.ds(base, bpw)])

    return k(table, idx)


def _sc_degree(row, n, d):
    e = row.shape[0]
    per = e // _NW
    cs = _chunk_size(per)
    nch = per // cs
    np_ = _pad_nodes(n)
    rpt = np_ // _NS
    zr = 8
    kg = d // _L

    @functools.partial(
        pl.kernel,
        out_type=jax.ShapeDtypeStruct((_NC * np_, d), jnp.float32),
        mesh=_sc_mesh(),
        scratch_types=[
            pltpu.VMEM((cs,), jnp.int32),
            pltpu.VMEM((cs, d), jnp.float32),
            pltpu.VMEM((zr, d), jnp.float32),
            pltpu.VMEM_SHARED((np_, d), jnp.float32),
        ],
    )
    def deg_kernel(row_hbm, out_hbm, rowv, onesv, zv, deg_sh):
        c = lax.axis_index("c")
        s = lax.axis_index("s")
        wid = c * _NS + s

        def fill(i, _):
            for k in range(kg):
                onesv[i, pl.ds(k * _L, _L)] = jnp.full((_L,), 1.0,
                                                       jnp.float32)
            return 0
        lax.fori_loop(0, cs, fill, 0)

        def fillz(i, _):
            for k in range(kg):
                zv[i, pl.ds(k * _L, _L)] = jnp.zeros((_L,), jnp.float32)
            return 0
        lax.fori_loop(0, zr, fillz, 0)

        def zloop(t, _):
            pltpu.sync_copy(zv, deg_sh.at[pl.ds(s * rpt + t * zr, zr)])
            return 0
        lax.fori_loop(0, rpt // zr, zloop, 0)
        plsc.subcore_barrier()

        def chunk(j, _):
            base = wid * per + j * cs
            pltpu.sync_copy(row_hbm.at[pl.ds(base, cs)], rowv)
            pltpu.sync_copy(onesv, deg_sh.at[rowv], add=True)
            return 0
        lax.fori_loop(0, nch, chunk, 0)
        plsc.subcore_barrier()
        pltpu.sync_copy(deg_sh.at[pl.ds(s * rpt, rpt)],
                        out_hbm.at[pl.ds(c * np_ + s * rpt, rpt)])

    return deg_kernel(row).reshape(_NC, np_, d)


def _sc_message(row, col, ee, h, dis):
    e = row.shape[0]
    n, d = h.shape
    kg = d // _L
    per = e // _NW
    cs = _chunk_size(per)
    nch = per // cs
    np_ = _pad_nodes(n)
    rpt = np_ // _NS
    zr = 8
    use_norm = dis is not None

    scratch = [
        pltpu.VMEM((cs,), jnp.int32),
        pltpu.VMEM((cs,), jnp.int32),
        pltpu.VMEM((cs, d), jnp.float32),
        pltpu.VMEM((cs, d), jnp.float32),
        pltpu.VMEM((zr, d), jnp.float32),
        pltpu.SemaphoreType.DMA,
        pltpu.VMEM_SHARED((np_, d), jnp.float32),
    ]
    if use_norm:
        scratch += [
            pltpu.VMEM((cs, d), jnp.float32),
        ]

    def body(*refs):
        if use_norm:
            (row_hbm, col_hbm, ee_hbm, h_hbm, dis_hbm, out_hbm,
             rowv, colv, hrows, eev, zv, sem, agg_sh, drv) = refs
        else:
            (row_hbm, col_hbm, ee_hbm, h_hbm, out_hbm,
             rowv, colv, hrows, eev, zv, sem, agg_sh) = refs
        c = lax.axis_index("c")
        s = lax.axis_index("s")
        wid = c * _NS + s

        def fillz(i, _):
            for k in range(kg):
                zv[i, pl.ds(k * _L, _L)] = jnp.zeros((_L,), jnp.float32)
            return 0
        lax.fori_loop(0, zr, fillz, 0)

        def zloop(t, _):
            pltpu.sync_copy(zv, agg_sh.at[pl.ds(s * rpt + t * zr, zr)])
            return 0
        lax.fori_loop(0, rpt // zr, zloop, 0)
        plsc.subcore_barrier()

        def chunk(j, _):
            base = wid * per + j * cs
            pltpu.sync_copy(row_hbm.at[pl.ds(base, cs)], rowv)
            pltpu.sync_copy(col_hbm.at[pl.ds(base, cs)], colv)
            gat = pltpu.async_copy(h_hbm.at[rowv], hrows, sem)
            if use_norm:
                gr = pltpu.async_copy(dis_hbm.at[rowv], drv, sem)
            pltpu.sync_copy(ee_hbm.at[pl.ds(base, cs)], eev)
            gat.wait()
            if use_norm:
                gr.wait()

            def edge(i, _):
                for k in range(kg):
                    v = (hrows[i, pl.ds(k * _L, _L)] +
                         eev[i, pl.ds(k * _L, _L)])
                    v = jnp.maximum(v, 0.0)
                    if use_norm:
                        v = v * drv[i, pl.ds(k * _L, _L)]
                    hrows[i, pl.ds(k * _L, _L)] = v
                return 0
            lax.fori_loop(0, cs, edge, 0, unroll=4)
            pltpu.sync_copy(hrows, agg_sh.at[colv], add=True)
            return 0
        lax.fori_loop(0, nch, chunk, 0)
        plsc.subcore_barrier()
        pltpu.sync_copy(agg_sh.at[pl.ds(s * rpt, rpt)],
                        out_hbm.at[pl.ds(c * np_ + s * rpt, rpt)])

    kern = functools.partial(
        pl.kernel,
        out_type=jax.ShapeDtypeStruct((_NC * np_, d), jnp.float32),
        mesh=_sc_mesh(),
        scratch_types=scratch,
    )(body)
    if use_norm:
        out = kern(row, col, ee, h, dis)
    else:
        out = kern(row, col, ee, h)
    return out.reshape(_NC, np_, d)


def _tc_edge_mlp(ea, w1, b1, w2, b2):
    e, de = ea.shape
    d = w1.shape[-1]
    be = 2000
    assert e % be == 0

    def body(ea_ref, w1_ref, b1_ref, w2_ref, b2_ref, o_ref):
        t = jnp.dot(ea_ref[...], w1_ref[...],
                    preferred_element_type=jnp.float32)
        t = jnp.maximum(t + b1_ref[...], 0.0)
        o_ref[...] = jnp.dot(t, w2_ref[...],
                             preferred_element_type=jnp.float32) + b2_ref[...]

    const = lambda i: (0, 0)
    return pl.pallas_call(
        body,
        grid=(e // be,),
        in_specs=[
            pl.BlockSpec((be, de), lambda i: (i, 0)),
            pl.BlockSpec((de, d), const),
            pl.BlockSpec((1, d), const),
            pl.BlockSpec((d, d), const),
            pl.BlockSpec((1, d), const),
        ],
        out_specs=pl.BlockSpec((be, d), lambda i: (i, 0)),
        out_shape=jax.ShapeDtypeStruct((e, d), jnp.float32),
    )(ea, w1, b1, w2, b2)


def _tc_linear(x, w, b):
    n, dn = x.shape
    d = w.shape[-1]
    bn = 2000
    assert n % bn == 0

    def body(x_ref, w_ref, b_ref, o_ref):
        o_ref[...] = jnp.dot(x_ref[...], w_ref[...],
                             preferred_element_type=jnp.float32) + b_ref[...]

    return pl.pallas_call(
        body,
        grid=(n // bn,),
        in_specs=[
            pl.BlockSpec((bn, dn), lambda i: (i, 0)),
            pl.BlockSpec((dn, d), lambda i: (0, 0)),
            pl.BlockSpec((1, d), lambda i: (0, 0)),
        ],
        out_specs=pl.BlockSpec((bn, d), lambda i: (i, 0)),
        out_shape=jax.ShapeDtypeStruct((n, d), jnp.float32),
    )(x, w, b)


def _tc_deg_combine(degp, n, d_feat):
    nc, np_, l = degp.shape

    def body(dp, deg_ref, dis_ref, disw_ref):
        d = jnp.zeros((np_, 1), jnp.float32)
        for c in range(nc):
            d = d + jnp.sum(dp[c], axis=1, keepdims=True)
        d = d[:n] * (1.0 / l) + 1.0
        deg_ref[...] = d
        r = lax.rsqrt(d)
        dis_ref[...] = r
        disw_ref[...] = jnp.broadcast_to(r, (n, d_feat))

    return pl.pallas_call(
        body,
        out_shape=[jax.ShapeDtypeStruct((n, 1), jnp.float32),
                   jax.ShapeDtypeStruct((n, 1), jnp.float32),
                   jax.ShapeDtypeStruct((n, d_feat), jnp.float32)],
    )(degp)


def _bn_relu(r, g_ref, b_ref):
    mu = jnp.mean(r, axis=0, keepdims=True)
    var = jnp.mean((r - mu) ** 2, axis=0, keepdims=True)
    return (r - mu) * lax.rsqrt(var + 1e-5) * g_ref[...] + b_ref[...]


def _tc_gcn_node(aggp, h0, deg, dis, root, g, b):
    n, d = h0.shape

    def body(aggp_ref, h0_ref, deg_ref, dis_ref, root_ref, g_ref, b_ref,
             o_ref):
        agg = (aggp_ref[0] + aggp_ref[1])[:n] * dis_ref[...]
        t = agg + jnp.maximum(h0_ref[...] + root_ref[...], 0.0) / deg_ref[...]
        r = jnp.maximum(t, 0.0)
        o_ref[...] = _bn_relu(r, g_ref, b_ref)

    return pl.pallas_call(
        body,
        out_shape=jax.ShapeDtypeStruct((n, d), jnp.float32),
    )(aggp, h0, deg, dis, root, g, b)


def _tc_gin_node(aggp, h, eps, w1, b1, w2, b2, g, b):
    n, d = h.shape

    def body(aggp_ref, h_ref, eps_ref, w1_ref, b1_ref, w2_ref, b2_ref,
             g_ref, b2g_ref, o_ref):
        z = (1.0 + eps_ref[0, 0]) * h_ref[...] + (aggp_ref[0] + aggp_ref[1])[:n]
        t = jnp.maximum(
            jnp.dot(z, w1_ref[...], preferred_element_type=jnp.float32)
            + b1_ref[...], 0.0)
        z = jnp.dot(t, w2_ref[...],
                    preferred_element_type=jnp.float32) + b2_ref[...]
        r = jnp.maximum(z, 0.0)
        o_ref[...] = _bn_relu(r, g_ref, b2g_ref)

    return pl.pallas_call(
        body,
        out_shape=jax.ShapeDtypeStruct((n, d), jnp.float32),
    )(aggp, h, eps, w1, b1, w2, b2, g, b)


def _tc_pool_head(h1, h2, h3, batch2d, ng, w1a, w1b, w1c, fb1, w4, b4):
    n, d = h1.shape
    out = w4.shape[-1]

    def body(h1_ref, h2_ref, h3_ref, bt_ref, w1a_ref, w1b_ref, w1c_ref,
             fb1_ref, w4_ref, b4_ref, o_ref):
        gid = lax.broadcasted_iota(jnp.int32, (n, ng), 1)
        bb = jnp.broadcast_to(bt_ref[...], (n, ng))
        p = (bb == gid).astype(jnp.float32)
        dims = (((0,), (0,)), ((), ()))
        cnt = lax.dot_general(p, jnp.ones((n, 1), jnp.float32), dims,
                              preferred_element_type=jnp.float32)
        inv = 1.0 / jnp.maximum(cnt, 1.0)
        acc = fb1_ref[...]
        for h_ref, w_ref in ((h1_ref, w1a_ref), (h2_ref, w1b_ref),
                             (h3_ref, w1c_ref)):
            pooled = lax.dot_general(p, h_ref[...], dims,
                                     preferred_element_type=jnp.float32)
            acc = acc + jnp.dot(pooled * inv, w_ref[...],
                                preferred_element_type=jnp.float32)
        o_ref[...] = jnp.dot(jnp.maximum(acc, 0.0), w4_ref[...],
                             preferred_element_type=jnp.float32) + b4_ref[...]

    return pl.pallas_call(
        body,
        out_shape=jax.ShapeDtypeStruct((ng, out), jnp.float32),
    )(h1, h2, h3, batch2d, w1a, w1b, w1c, fb1, w4, b4)


def kernel(x, edge_index, edge_attr, batch, params):
    p = params
    n, _ = x.shape
    h = p["gcn_lin_w"].shape[-1]
    row = edge_index[0]
    col = edge_index[1]

    r1 = lambda a: a.reshape(1, -1)
    ee0 = _tc_edge_mlp(edge_attr, p["gcn_be1_w"], r1(p["gcn_be1_b"]),
                       p["gcn_be2_w"], r1(p["gcn_be2_b"]))
    ee1 = _tc_edge_mlp(edge_attr, p["gin1_be1_w"], r1(p["gin1_be1_b"]),
                       p["gin1_be2_w"], r1(p["gin1_be2_b"]))
    ee2 = _tc_edge_mlp(edge_attr, p["gin2_be1_w"], r1(p["gin2_be1_b"]),
                       p["gin2_be2_w"], r1(p["gin2_be2_b"]))
    h0 = _tc_linear(x, p["gcn_lin_w"], r1(p["gcn_lin_b"]))

    degp = _sc_degree(row, n, h)
    deg, dis, disw = _tc_deg_combine(degp, n, h)

    aggp = _sc_message(row, col, ee0, h0, disw)
    h1 = _tc_gcn_node(aggp, h0, deg, dis, p["gcn_root"],
                      r1(p["bn0_g"]), r1(p["bn0_b"]))

    aggp = _sc_message(row, col, ee1, h1, None)
    h2 = _tc_gin_node(aggp, h1, p["gin1_eps"].reshape(1, 1),
                      p["gin1_mlp1_w"], r1(p["gin1_mlp1_b"]),
                      p["gin1_mlp2_w"], r1(p["gin1_mlp2_b"]),
                      r1(p["bn1_g"]), r1(p["bn1_b"]))

    aggp = _sc_message(row, col, ee2, h2, None)
    h3 = _tc_gin_node(aggp, h2, p["gin2_eps"].reshape(1, 1),
                      p["gin2_mlp1_w"], r1(p["gin2_mlp1_b"]),
                      p["gin2_mlp2_w"], r1(p["gin2_mlp2_b"]),
                      r1(p["bn2_g"]), r1(p["bn2_b"]))

    fw = p["fc1_w"]
    ng = 64
    out = _tc_pool_head(
        h1, h2, h3, batch.reshape(n, 1), ng,
        fw[:h], fw[h:2 * h], fw[2 * h:], r1(p["fc1_b"]),
        p["fc4_w"], r1(p["fc4_b"]))
    return out


# speedup vs baseline: 1.5297x; 1.5297x over previous
import functools

import jax
import jax.numpy as jnp
from jax import lax
from jax.experimental import pallas as pl
from jax.experimental.pallas import tpu as pltpu
from jax.experimental.pallas import tpu_sc as plsc

_NC = 2
_NS = 16
_L = 16
_NW = _NC * _NS


def _sc_mesh():
    return plsc.VectorSubcoreMesh(
        core_axis_name="c", subcore_axis_name="s",
        num_cores=_NC, num_subcores=_NS)


def _chunk_size(per_tile):
    for c in range(128, 0, -16):
        if per_tile % c == 0:
            return c
    raise ValueError(f"per-tile edge count {per_tile} not divisible by 16")


def _pad_nodes(n):
    blk = _NS * 8
    return ((n + blk - 1) // blk) * blk


def _sc_degree(row, n, d):
    e = row.shape[0]
    per = e // _NW
    cs = _chunk_size(per)
    nch = per // cs
    np_ = _pad_nodes(n)
    rpt = np_ // _NS
    zr = 8
    kg = d // _L

    @functools.partial(
        pl.kernel,
        out_type=jax.ShapeDtypeStruct((_NC * np_, d), jnp.float32),
        mesh=_sc_mesh(),
        scratch_types=[
            pltpu.VMEM((cs,), jnp.int32),
            pltpu.VMEM((cs, d), jnp.float32),
            pltpu.VMEM((zr, d), jnp.float32),
            pltpu.VMEM_SHARED((np_, d), jnp.float32),
        ],
    )
    def deg_kernel(row_hbm, out_hbm, rowv, onesv, zv, deg_sh):
        c = lax.axis_index("c")
        s = lax.axis_index("s")
        wid = c * _NS + s

        def fill(i, _):
            for k in range(kg):
                onesv[i, pl.ds(k * _L, _L)] = jnp.full((_L,), 1.0,
                                                       jnp.float32)
            return 0
        lax.fori_loop(0, cs, fill, 0)

        def fillz(i, _):
            for k in range(kg):
                zv[i, pl.ds(k * _L, _L)] = jnp.zeros((_L,), jnp.float32)
            return 0
        lax.fori_loop(0, zr, fillz, 0)

        def zloop(t, _):
            pltpu.sync_copy(zv, deg_sh.at[pl.ds(s * rpt + t * zr, zr)])
            return 0
        lax.fori_loop(0, rpt // zr, zloop, 0)
        plsc.subcore_barrier()

        def chunk(j, _):
            base = wid * per + j * cs
            pltpu.sync_copy(row_hbm.at[pl.ds(base, cs)], rowv)
            pltpu.sync_copy(onesv, deg_sh.at[rowv], add=True)
            return 0
        lax.fori_loop(0, nch, chunk, 0)
        plsc.subcore_barrier()
        pltpu.sync_copy(deg_sh.at[pl.ds(s * rpt, rpt)],
                        out_hbm.at[pl.ds(c * np_ + s * rpt, rpt)])

    return deg_kernel(row).reshape(_NC, np_, d)


def _sc_message(row, col, ee, h, dis):
    e = row.shape[0]
    n, d = h.shape
    kg = d // _L
    per = e // _NW
    cs = _chunk_size(per)
    nch = per // cs
    np_ = _pad_nodes(n)
    rpt = np_ // _NS
    zr = 8
    use_norm = dis is not None

    scratch = [
        pltpu.VMEM((cs,), jnp.int32),
        pltpu.VMEM((cs,), jnp.int32),
        pltpu.VMEM((cs, d), jnp.float32),
        pltpu.VMEM((cs, d), jnp.float32),
        pltpu.VMEM((zr, d), jnp.float32),
        pltpu.SemaphoreType.DMA,
        pltpu.VMEM_SHARED((np_, d), jnp.float32),
    ]
    if use_norm:
        scratch += [
            pltpu.VMEM((cs, d), jnp.float32),
        ]

    def body(*refs):
        if use_norm:
            (row_hbm, col_hbm, ee_hbm, h_hbm, dis_hbm, out_hbm,
             rowv, colv, hrows, eev, zv, sem, agg_sh, drv) = refs
        else:
            (row_hbm, col_hbm, ee_hbm, h_hbm, out_hbm,
             rowv, colv, hrows, eev, zv, sem, agg_sh) = refs
        c = lax.axis_index("c")
        s = lax.axis_index("s")
        wid = c * _NS + s

        def fillz(i, _):
            for k in range(kg):
                zv[i, pl.ds(k * _L, _L)] = jnp.zeros((_L,), jnp.float32)
            return 0
        lax.fori_loop(0, zr, fillz, 0)

        def zloop(t, _):
            pltpu.sync_copy(zv, agg_sh.at[pl.ds(s * rpt + t * zr, zr)])
            return 0
        lax.fori_loop(0, rpt // zr, zloop, 0)
        plsc.subcore_barrier()

        def chunk(j, _):
            base = wid * per + j * cs
            pltpu.sync_copy(row_hbm.at[pl.ds(base, cs)], rowv)
            pltpu.sync_copy(col_hbm.at[pl.ds(base, cs)], colv)
            gat = pltpu.async_copy(h_hbm.at[rowv], hrows, sem)
            if use_norm:
                gr = pltpu.async_copy(dis_hbm.at[rowv], drv, sem)
            pltpu.sync_copy(ee_hbm.at[pl.ds(base, cs)], eev)
            gat.wait()
            if use_norm:
                gr.wait()

            def edge(i, _):
                for k in range(kg):
                    v = (hrows[i, pl.ds(k * _L, _L)] +
                         eev[i, pl.ds(k * _L, _L)])
                    v = jnp.maximum(v, 0.0)
                    if use_norm:
                        v = v * drv[i, pl.ds(k * _L, _L)]
                    hrows[i, pl.ds(k * _L, _L)] = v
                return 0
            lax.fori_loop(0, cs, edge, 0)
            pltpu.sync_copy(hrows, agg_sh.at[colv], add=True)
            return 0
        lax.fori_loop(0, nch, chunk, 0)
        plsc.subcore_barrier()
        pltpu.sync_copy(agg_sh.at[pl.ds(s * rpt, rpt)],
                        out_hbm.at[pl.ds(c * np_ + s * rpt, rpt)])

    kern = functools.partial(
        pl.kernel,
        out_type=jax.ShapeDtypeStruct((_NC * np_, d), jnp.float32),
        mesh=_sc_mesh(),
        scratch_types=scratch,
    )(body)
    if use_norm:
        out = kern(row, col, ee, h, dis)
    else:
        out = kern(row, col, ee, h)
    return out.reshape(_NC, np_, d)


def _tc_edge_mlp(ea, w1, b1, w2, b2):
    e, de = ea.shape
    d = w1.shape[-1]
    be = 2000
    assert e % be == 0

    def body(ea_ref, w1_ref, b1_ref, w2_ref, b2_ref, o_ref):
        t = jnp.dot(ea_ref[...], w1_ref[...],
                    preferred_element_type=jnp.float32)
        t = jnp.maximum(t + b1_ref[...], 0.0)
        o_ref[...] = jnp.dot(t, w2_ref[...],
                             preferred_element_type=jnp.float32) + b2_ref[...]

    const = lambda i: (0, 0)
    return pl.pallas_call(
        body,
        grid=(e // be,),
        in_specs=[
            pl.BlockSpec((be, de), lambda i: (i, 0)),
            pl.BlockSpec((de, d), const),
            pl.BlockSpec((1, d), const),
            pl.BlockSpec((d, d), const),
            pl.BlockSpec((1, d), const),
        ],
        out_specs=pl.BlockSpec((be, d), lambda i: (i, 0)),
        out_shape=jax.ShapeDtypeStruct((e, d), jnp.float32),
    )(ea, w1, b1, w2, b2)


def _tc_linear(x, w, b):
    n, dn = x.shape
    d = w.shape[-1]
    bn = 2000
    assert n % bn == 0

    def body(x_ref, w_ref, b_ref, o_ref):
        o_ref[...] = jnp.dot(x_ref[...], w_ref[...],
                             preferred_element_type=jnp.float32) + b_ref[...]

    return pl.pallas_call(
        body,
        grid=(n // bn,),
        in_specs=[
            pl.BlockSpec((bn, dn), lambda i: (i, 0)),
            pl.BlockSpec((dn, d), lambda i: (0, 0)),
            pl.BlockSpec((1, d), lambda i: (0, 0)),
        ],
        out_specs=pl.BlockSpec((bn, d), lambda i: (i, 0)),
        out_shape=jax.ShapeDtypeStruct((n, d), jnp.float32),
    )(x, w, b)


def _tc_deg_combine(degp, n, d_feat):
    nc, np_, l = degp.shape

    def body(dp, deg_ref, dis_ref, disw_ref):
        d = jnp.zeros((np_, 1), jnp.float32)
        for c in range(nc):
            d = d + jnp.sum(dp[c], axis=1, keepdims=True)
        d = d[:n] * (1.0 / l) + 1.0
        deg_ref[...] = d
        r = lax.rsqrt(d)
        dis_ref[...] = r
        disw_ref[...] = jnp.broadcast_to(r, (n, d_feat))

    return pl.pallas_call(
        body,
        out_shape=[jax.ShapeDtypeStruct((n, 1), jnp.float32),
                   jax.ShapeDtypeStruct((n, 1), jnp.float32),
                   jax.ShapeDtypeStruct((n, d_feat), jnp.float32)],
    )(degp)


def _bn_relu(r, g_ref, b_ref):
    mu = jnp.mean(r, axis=0, keepdims=True)
    var = jnp.mean((r - mu) ** 2, axis=0, keepdims=True)
    return (r - mu) * lax.rsqrt(var + 1e-5) * g_ref[...] + b_ref[...]


def _tc_gcn_node(aggp, h0, deg, dis, root, g, b):
    n, d = h0.shape

    def body(aggp_ref, h0_ref, deg_ref, dis_ref, root_ref, g_ref, b_ref,
             o_ref):
        agg = (aggp_ref[0] + aggp_ref[1])[:n] * dis_ref[...]
        t = agg + jnp.maximum(h0_ref[...] + root_ref[...], 0.0) / deg_ref[...]
        r = jnp.maximum(t, 0.0)
        o_ref[...] = _bn_relu(r, g_ref, b_ref)

    return pl.pallas_call(
        body,
        out_shape=jax.ShapeDtypeStruct((n, d), jnp.float32),
    )(aggp, h0, deg, dis, root, g, b)


def _tc_gin_node(aggp, h, eps, w1, b1, w2, b2, g, b):
    n, d = h.shape

    def body(aggp_ref, h_ref, eps_ref, w1_ref, b1_ref, w2_ref, b2_ref,
             g_ref, b2g_ref, o_ref):
        z = (1.0 + eps_ref[0, 0]) * h_ref[...] + (aggp_ref[0] + aggp_ref[1])[:n]
        t = jnp.maximum(
            jnp.dot(z, w1_ref[...], preferred_element_type=jnp.float32)
            + b1_ref[...], 0.0)
        z = jnp.dot(t, w2_ref[...],
                    preferred_element_type=jnp.float32) + b2_ref[...]
        r = jnp.maximum(z, 0.0)
        o_ref[...] = _bn_relu(r, g_ref, b2g_ref)

    return pl.pallas_call(
        body,
        out_shape=jax.ShapeDtypeStruct((n, d), jnp.float32),
    )(aggp, h, eps, w1, b1, w2, b2, g, b)


def _tc_pool_head(h1, h2, h3, batch2d, ng, w1a, w1b, w1c, fb1, w4, b4):
    n, d = h1.shape
    out = w4.shape[-1]

    def body(h1_ref, h2_ref, h3_ref, bt_ref, w1a_ref, w1b_ref, w1c_ref,
             fb1_ref, w4_ref, b4_ref, o_ref):
        gid = lax.broadcasted_iota(jnp.int32, (n, ng), 1)
        bb = jnp.broadcast_to(bt_ref[...], (n, ng))
        p = (bb == gid).astype(jnp.float32)
        dims = (((0,), (0,)), ((), ()))
        cnt = lax.dot_general(p, jnp.ones((n, 1), jnp.float32), dims,
                              preferred_element_type=jnp.float32)
        inv = 1.0 / jnp.maximum(cnt, 1.0)
        acc = fb1_ref[...]
        for h_ref, w_ref in ((h1_ref, w1a_ref), (h2_ref, w1b_ref),
                             (h3_ref, w1c_ref)):
            pooled = lax.dot_general(p, h_ref[...], dims,
                                     preferred_element_type=jnp.float32)
            acc = acc + jnp.dot(pooled * inv, w_ref[...],
                                preferred_element_type=jnp.float32)
        o_ref[...] = jnp.dot(jnp.maximum(acc, 0.0), w4_ref[...],
                             preferred_element_type=jnp.float32) + b4_ref[...]

    return pl.pallas_call(
        body,
        out_shape=jax.ShapeDtypeStruct((ng, out), jnp.float32),
    )(h1, h2, h3, batch2d, w1a, w1b, w1c, fb1, w4, b4)


def kernel(x, edge_index, edge_attr, batch, params):
    p = params
    n, _ = x.shape
    h = p["gcn_lin_w"].shape[-1]
    row = edge_index[0]
    col = edge_index[1]

    r1 = lambda a: a.reshape(1, -1)
    ee0 = _tc_edge_mlp(edge_attr, p["gcn_be1_w"], r1(p["gcn_be1_b"]),
                       p["gcn_be2_w"], r1(p["gcn_be2_b"]))
    ee1 = _tc_edge_mlp(edge_attr, p["gin1_be1_w"], r1(p["gin1_be1_b"]),
                       p["gin1_be2_w"], r1(p["gin1_be2_b"]))
    ee2 = _tc_edge_mlp(edge_attr, p["gin2_be1_w"], r1(p["gin2_be1_b"]),
                       p["gin2_be2_w"], r1(p["gin2_be2_b"]))
    h0 = _tc_linear(x, p["gcn_lin_w"], r1(p["gcn_lin_b"]))

    degp = _sc_degree(row, n, h)
    deg, dis, disw = _tc_deg_combine(degp, n, h)

    aggp = _sc_message(row, col, ee0, h0, disw)
    h1 = _tc_gcn_node(aggp, h0, deg, dis, p["gcn_root"],
                      r1(p["bn0_g"]), r1(p["bn0_b"]))

    aggp = _sc_message(row, col, ee1, h1, None)
    h2 = _tc_gin_node(aggp, h1, p["gin1_eps"].reshape(1, 1),
                      p["gin1_mlp1_w"], r1(p["gin1_mlp1_b"]),
                      p["gin1_mlp2_w"], r1(p["gin1_mlp2_b"]),
                      r1(p["bn1_g"]), r1(p["bn1_b"]))

    aggp = _sc_message(row, col, ee2, h2, None)
    h3 = _tc_gin_node(aggp, h2, p["gin2_eps"].reshape(1, 1),
                      p["gin2_mlp1_w"], r1(p["gin2_mlp1_b"]),
                      p["gin2_mlp2_w"], r1(p["gin2_mlp2_b"]),
                      r1(p["bn2_g"]), r1(p["bn2_b"]))

    fw = p["fc1_w"]
    ng = 64
    out = _tc_pool_head(
        h1, h2, h3, batch.reshape(n, 1), ng,
        fw[:h], fw[h:2 * h], fw[2 * h:], r1(p["fc1_b"]),
        p["fc4_w"], r1(p["fc4_b"]))
    return out
